# initial kernel scaffold (unmeasured)
import jax
import jax.numpy as jnp
from jax import lax
from jax.experimental import pallas as pl
from jax.experimental.pallas import tpu as pltpu


def kernel(
    x,
):
    def body(*refs):
        pass

    out_shape = jax.ShapeDtypeStruct(..., jnp.float32)
    return pl.pallas_call(body, out_shape=out_shape)(...)


# baseline (device time: 58898 ns/iter reference)
import jax
import jax.numpy as jnp
from jax import lax
from jax.experimental import pallas as pl
from jax.experimental.pallas import tpu as pltpu

N_DEV = 8


def kernel(x):
    m_per, n = x.shape

    def body(x_ref, out_ref, comm_ref, send_sems, recv_sems):
        my = lax.axis_index("i")
        left = lax.rem(my + (N_DEV - 1), N_DEV)
        right = lax.rem(my + 1, N_DEV)

        barrier_sem = pltpu.get_barrier_semaphore()
        for nbr in (left, right):
            pl.semaphore_signal(
                barrier_sem, inc=1,
                device_id=(nbr,), device_id_type=pl.DeviceIdType.MESH,
            )
        pl.semaphore_wait(barrier_sem, 2)

        comm_ref[0] = x_ref[...].astype(jnp.bfloat16)
        out_ref[pl.ds(my * m_per, m_per), :] = comm_ref[0]

        for h in range(N_DEV - 1):
            rdma = pltpu.make_async_remote_copy(
                src_ref=comm_ref.at[h],
                dst_ref=comm_ref.at[h + 1],
                send_sem=send_sems.at[h],
                recv_sem=recv_sems.at[h],
                device_id=(right,),
                device_id_type=pl.DeviceIdType.MESH,
            )
            rdma.start()
            rdma.wait()
            origin = lax.rem(my + (N_DEV - 1 - h), N_DEV)
            out_ref[pl.ds(origin * m_per, m_per), :] = comm_ref[h + 1]

    return pl.pallas_call(
        body,
        out_shape=jax.ShapeDtypeStruct((N_DEV * m_per, n), jnp.bfloat16),
        in_specs=[pl.BlockSpec(memory_space=pltpu.VMEM)],
        out_specs=pl.BlockSpec(memory_space=pltpu.VMEM),
        scratch_shapes=[
            pltpu.VMEM((N_DEV, m_per, n), jnp.bfloat16),
            pltpu.SemaphoreType.DMA((N_DEV - 1,)),
            pltpu.SemaphoreType.DMA((N_DEV - 1,)),
        ],
        compiler_params=pltpu.CompilerParams(collective_id=0),
    )(x)


# device time: 26789 ns/iter; 2.1986x vs baseline; 2.1986x over previous
import jax
import jax.numpy as jnp
from jax import lax
from jax.experimental import pallas as pl
from jax.experimental.pallas import tpu as pltpu

N_DEV = 8

DIMS = ((1, 3, 4), (3, 4, 1), (4, 1, 3))

M_PARTS = (192, 192, 128)
M_OFFS = (0, 192, 384)


def _gray(dims):
    a, b, c = dims
    return (0, a, b, b ^ a, c, c ^ a, c ^ b, c ^ b ^ a)


def kernel(x):
    m_per, n = x.shape
    assert m_per == sum(M_PARTS)

    def body(x_ref, out_ref, comm0, comm1, comm2, send_sems, recv_sems):
        comms = (comm0, comm1, comm2)
        my = lax.axis_index("i")

        barrier_sem = pltpu.get_barrier_semaphore()
        for mask in (1, 3, 4):
            pl.semaphore_signal(
                barrier_sem, inc=1,
                device_id=(my ^ mask,), device_id_type=pl.DeviceIdType.MESH,
            )
        pl.semaphore_wait(barrier_sem, 3)

        for j in range(3):
            comms[j][0] = x_ref[
                M_OFFS[j]:M_OFFS[j] + M_PARTS[j], :
            ].astype(jnp.bfloat16)

        def make_rdma(j, k):
            return pltpu.make_async_remote_copy(
                src_ref=comms[j].at[pl.ds(0, 2 ** k)],
                dst_ref=comms[j].at[pl.ds(2 ** k, 2 ** k)],
                send_sem=send_sems.at[j, k],
                recv_sem=recv_sems.at[j, k],
                device_id=(my ^ DIMS[j][k],),
                device_id_type=pl.DeviceIdType.MESH,
            )

        def store(j, r):
            origin = my ^ _gray(DIMS[j])[r]
            out_ref[pl.ds(origin * m_per + M_OFFS[j], M_PARTS[j]), :] = (
                comms[j][r]
            )

        rdmas = [make_rdma(j, 0) for j in range(3)]
        for r in rdmas:
            r.start()
        for j in range(3):
            store(j, 0)

        for k in (1, 2):
            next_rdmas = []
            for j in range(3):
                rdmas[j].wait()
                nxt = make_rdma(j, k)
                nxt.start()
                next_rdmas.append(nxt)
                for r in range(2 ** (k - 1), 2 ** k):
                    store(j, r)
            rdmas = next_rdmas

        for j in range(3):
            rdmas[j].wait()
            for r in range(4, 8):
                store(j, r)

    return pl.pallas_call(
        body,
        out_shape=jax.ShapeDtypeStruct((N_DEV * m_per, n), jnp.bfloat16),
        in_specs=[pl.BlockSpec(memory_space=pltpu.VMEM)],
        out_specs=pl.BlockSpec(memory_space=pltpu.VMEM),
        scratch_shapes=[
            pltpu.VMEM((N_DEV, M_PARTS[0], n), jnp.bfloat16),
            pltpu.VMEM((N_DEV, M_PARTS[1], n), jnp.bfloat16),
            pltpu.VMEM((N_DEV, M_PARTS[2], n), jnp.bfloat16),
            pltpu.SemaphoreType.DMA((3, 3)),
            pltpu.SemaphoreType.DMA((3, 3)),
        ],
        compiler_params=pltpu.CompilerParams(collective_id=0),
    )(x)


# device time: 25765 ns/iter; 2.2860x vs baseline; 1.0397x over previous
import jax
import jax.numpy as jnp
from jax import lax
from jax.experimental import pallas as pl
from jax.experimental.pallas import tpu as pltpu

N_DEV = 8

DIMS = ((1, 3, 4), (3, 4, 1), (4, 1, 3))

M_PARTS = (176, 176, 160)
M_OFFS = (0, 176, 352)


def _gray(dims):
    a, b, c = dims
    return (0, a, b, b ^ a, c, c ^ a, c ^ b, c ^ b ^ a)


def kernel(x):
    m_per, n = x.shape
    assert m_per == sum(M_PARTS)

    def body(x_ref, out_ref, send_sems, recv_sems):
        my = lax.axis_index("i")

        barrier_sem = pltpu.get_barrier_semaphore()
        for mask in (1, 3, 4):
            pl.semaphore_signal(
                barrier_sem, inc=1,
                device_id=(my ^ mask,), device_id_type=pl.DeviceIdType.MESH,
            )
        for j in range(3):
            out_ref[pl.ds(my * m_per + M_OFFS[j], M_PARTS[j]), :] = x_ref[
                M_OFFS[j]:M_OFFS[j] + M_PARTS[j], :
            ].astype(jnp.bfloat16)
        pl.semaphore_wait(barrier_sem, 3)

        def block(j, origin):
            return out_ref.at[pl.ds(origin * m_per + M_OFFS[j], M_PARTS[j])]

        rd = [[None] * N_DEV for _ in range(3)]
        for k in range(3):
            for j in range(3):
                G = _gray(DIMS[j])
                for s in range(2 ** (k - 1), 2 ** k) if k else ():
                    rd[j][s].wait_recv()
                partner = my ^ DIMS[j][k]
                for r in range(2 ** k):
                    s = 2 ** k + r
                    origin = my ^ G[r]
                    rd[j][s] = pltpu.make_async_remote_copy(
                        src_ref=block(j, origin),
                        dst_ref=block(j, origin),
                        send_sem=send_sems.at[j, s],
                        recv_sem=recv_sems.at[j, s],
                        device_id=(partner,),
                        device_id_type=pl.DeviceIdType.MESH,
                    )
                    rd[j][s].start()

        for j in range(3):
            for s in range(4, 8):
                rd[j][s].wait_recv()
        for j in range(3):
            for s in range(1, 8):
                rd[j][s].wait_send()

    return pl.pallas_call(
        body,
        out_shape=jax.ShapeDtypeStruct((N_DEV * m_per, n), jnp.bfloat16),
        in_specs=[pl.BlockSpec(memory_space=pltpu.VMEM)],
        out_specs=pl.BlockSpec(memory_space=pltpu.VMEM),
        scratch_shapes=[
            pltpu.SemaphoreType.DMA((3, N_DEV)),
            pltpu.SemaphoreType.DMA((3, N_DEV)),
        ],
        compiler_params=pltpu.CompilerParams(collective_id=0),
    )(x)


# device time: 24255 ns/iter; 2.4283x vs baseline; 1.0623x over previous
import jax
import jax.numpy as jnp
from jax import lax
from jax.experimental import pallas as pl
from jax.experimental.pallas import tpu as pltpu

N_DEV = 8

DIMS = ((1, 3, 4), (3, 4, 1), (4, 1, 3))

M_PARTS = (176, 176, 160)
M_OFFS = (0, 176, 352)


def _gray(dims):
    a, b, c = dims
    return (0, a, b, b ^ a, c, c ^ a, c ^ b, c ^ b ^ a)


def kernel(x):
    m_per, n = x.shape
    assert m_per == sum(M_PARTS)

    def body(x_ref, out_ref, send_sems, recv_sems):
        my = lax.axis_index("i")

        barrier_sem = pltpu.get_barrier_semaphore()
        for mask in (1, 3, 4):
            pl.semaphore_signal(
                barrier_sem, inc=1,
                device_id=(my ^ mask,), device_id_type=pl.DeviceIdType.MESH,
            )
        for j in range(3):
            out_ref[pl.ds(my * m_per + M_OFFS[j], M_PARTS[j]), :] = x_ref[
                M_OFFS[j]:M_OFFS[j] + M_PARTS[j], :
            ].astype(jnp.bfloat16)
        pl.semaphore_wait(barrier_sem, 3)

        def block(j, origin):
            return out_ref.at[pl.ds(origin * m_per + M_OFFS[j], M_PARTS[j])]

        rd = [[None] * N_DEV for _ in range(3)]

        def start(j, s, k):
            G = _gray(DIMS[j])
            origin = my ^ G[s - 2 ** k]
            rd[j][s] = pltpu.make_async_remote_copy(
                src_ref=block(j, origin),
                dst_ref=block(j, origin),
                send_sem=send_sems.at[j, s],
                recv_sem=recv_sems.at[j, s],
                device_id=(my ^ DIMS[j][k],),
                device_id_type=pl.DeviceIdType.MESH,
            )
            rd[j][s].start()

        for j in range(3):
            start(j, 1, 0)
        for j in range(3):
            start(j, 2, 1)
        for j in range(3):
            start(j, 4, 2)
        for j in range(3):
            rd[j][1].wait_recv()
            start(j, 3, 1)
            start(j, 5, 2)
        for j in range(3):
            rd[j][2].wait_recv()
            rd[j][3].wait_recv()
            start(j, 6, 2)
            start(j, 7, 2)

        for j in range(3):
            for s in range(4, 8):
                rd[j][s].wait_recv()
        for j in range(3):
            for s in range(1, 8):
                rd[j][s].wait_send()

    return pl.pallas_call(
        body,
        out_shape=jax.ShapeDtypeStruct((N_DEV * m_per, n), jnp.bfloat16),
        in_specs=[pl.BlockSpec(memory_space=pltpu.VMEM)],
        out_specs=pl.BlockSpec(memory_space=pltpu.VMEM),
        scratch_shapes=[
            pltpu.SemaphoreType.DMA((3, N_DEV)),
            pltpu.SemaphoreType.DMA((3, N_DEV)),
        ],
        compiler_params=pltpu.CompilerParams(collective_id=0),
    )(x)
